# baseline (speedup 1.0000x reference)
import functools

import jax
import jax.numpy as jnp
from jax import lax
from jax.experimental import pallas as pl
from jax.experimental.pallas import tpu as pltpu

_VMEM_LIMIT = 48 * 1024 * 1024


def _fused_kernel(x_hbm, w_ref, y_ref, sub_ref, grads_ref, logits_ref,
                  x_buf, selx_ref, selc_ref, x_sem, *, feat_dim, chunk):
    j = pl.program_id(1)

    def x_dma(t):
        row = pl.multiple_of(t * chunk, chunk)
        return pltpu.make_async_copy(
            x_hbm.at[pl.ds(row, chunk), :], x_buf.at[t], x_sem.at[t])

    nsteps = x_buf.shape[0]

    @pl.when(j == 0)
    def _init():
        for t in range(nsteps):
            x_dma(t).start()
        idx = sub_ref[0:1, :]
        d_k = idx % feat_dim
        c_k = idx // feat_dim
        kshape = selx_ref.shape
        d_iota = lax.broadcasted_iota(jnp.int32, kshape, 0)
        selx_ref[...] = (d_iota == jnp.broadcast_to(d_k, kshape)).astype(
            jnp.bfloat16)
        cshape = selc_ref.shape
        c_iota = lax.broadcasted_iota(jnp.int32, cshape, 0)
        selc_ref[...] = (c_iota == jnp.broadcast_to(c_k, cshape)).astype(
            jnp.bfloat16)

    x_dma(j).wait()
    x = x_buf[j]

    logits = lax.dot_general(x, w_ref[...], (((1,), (1,)), ((), ())),
                             preferred_element_type=jnp.float32)
    m = jnp.max(logits, axis=-1, keepdims=True)
    e = jnp.exp(logits - m)
    s = jnp.sum(e, axis=-1, keepdims=True)
    p = e * (1.0 / s)

    yshape = logits.shape
    cls = lax.broadcasted_iota(jnp.int32, yshape, 1)
    y1h = (cls == jnp.broadcast_to(y_ref[...], yshape)).astype(jnp.float32)
    diff = p - y1h

    xg = jnp.dot(x.astype(jnp.bfloat16), selx_ref[...],
                 preferred_element_type=jnp.float32)
    dg = jnp.dot(diff.astype(jnp.bfloat16), selc_ref[...],
                 preferred_element_type=jnp.float32)
    grads_ref[...] = xg * dg
    logits_ref[...] = logits


def kernel(x_flat, w, y_labels, sub_idx):
    N, D = x_flat.shape
    C = w.shape[0]
    cutoff = int(sub_idx.shape[0])

    chunk = next(c for c in (512, 256, 128, 64, 32, 16, 8)
                 if N % c == 0)
    nb = N // chunk
    grid = (1, nb)
    nbj = grid[1]

    sub2d = jnp.broadcast_to(sub_idx.reshape(1, cutoff), (8, cutoff))
    y2d = y_labels.reshape(N, 1)

    grads, logits = pl.pallas_call(
        functools.partial(_fused_kernel, feat_dim=D, chunk=chunk),
        out_shape=(jax.ShapeDtypeStruct((N, cutoff), jnp.float32),
                   jax.ShapeDtypeStruct((N, C), jnp.float32)),
        grid_spec=pltpu.PrefetchScalarGridSpec(
            num_scalar_prefetch=0,
            grid=grid,
            in_specs=[
                pl.BlockSpec(memory_space=pl.ANY),
                pl.BlockSpec((C, D), lambda i, j: (0, 0)),
                pl.BlockSpec((chunk, 1), lambda i, j: (i * nbj + j, 0)),
                pl.BlockSpec((8, cutoff), lambda i, j: (0, 0)),
            ],
            out_specs=[
                pl.BlockSpec((chunk, cutoff), lambda i, j: (i * nbj + j, 0)),
                pl.BlockSpec((chunk, C), lambda i, j: (i * nbj + j, 0)),
            ],
            scratch_shapes=[pltpu.VMEM((nb, chunk, D), jnp.float32),
                            pltpu.VMEM((D, cutoff), jnp.bfloat16),
                            pltpu.VMEM((C, cutoff), jnp.bfloat16),
                            pltpu.SemaphoreType.DMA((nb,))]),
        compiler_params=pltpu.CompilerParams(
            dimension_semantics=("parallel", "arbitrary"),
            vmem_limit_bytes=_VMEM_LIMIT),
    )(x_flat, w, y2d, sub2d)
    return grads, logits


# speedup vs baseline: 1.0290x; 1.0290x over previous
import functools

import jax
import jax.numpy as jnp
from jax import lax
from jax.experimental import pallas as pl
from jax.experimental.pallas import tpu as pltpu

_VMEM_LIMIT = 48 * 1024 * 1024


def _fused_kernel(x_hbm, w_ref, y_ref, sub_ref, grads_ref, logits_ref,
                  x_buf, selx_ref, selc_ref, x_sem, *, feat_dim, chunk):
    j = pl.program_id(1)

    def x_dma(t):
        row = pl.multiple_of(t * chunk, chunk)
        return pltpu.make_async_copy(
            x_hbm.at[pl.ds(row, chunk), :], x_buf.at[t], x_sem.at[t])

    nsteps = x_buf.shape[0]

    @pl.when(j == 0)
    def _init():
        for t in range(nsteps):
            x_dma(t).start()
        idx = sub_ref[0:1, :]
        d_k = idx % feat_dim
        c_k = idx // feat_dim
        kshape = selx_ref.shape
        d_iota = lax.broadcasted_iota(jnp.int32, kshape, 0)
        selx_ref[...] = (d_iota == jnp.broadcast_to(d_k, kshape)).astype(
            jnp.bfloat16)
        cshape = selc_ref.shape
        c_iota = lax.broadcasted_iota(jnp.int32, cshape, 0)
        selc_ref[...] = (c_iota == jnp.broadcast_to(c_k, cshape)).astype(
            jnp.bfloat16)

    x_dma(j).wait()
    x = x_buf[j]

    logits = lax.dot_general(x, w_ref[...], (((1,), (1,)), ((), ())),
                             preferred_element_type=jnp.float32)
    m = jnp.max(logits, axis=-1, keepdims=True)
    e = jnp.exp(logits - m)
    s = jnp.sum(e, axis=-1, keepdims=True)
    p = e * (1.0 / s)

    yshape = logits.shape
    cls = lax.broadcasted_iota(jnp.int32, yshape, 1)
    y1h = (cls == jnp.broadcast_to(y_ref[...], yshape)).astype(jnp.float32)
    diff = p - y1h

    xg = jnp.dot(x.astype(jnp.bfloat16), selx_ref[...],
                 preferred_element_type=jnp.float32)
    dg = jnp.dot(diff.astype(jnp.bfloat16), selc_ref[...],
                 preferred_element_type=jnp.float32)
    grads_ref[...] = xg * dg
    logits_ref[...] = logits


def kernel(x_flat, w, y_labels, sub_idx):
    N, D = x_flat.shape
    C = w.shape[0]
    cutoff = int(sub_idx.shape[0])

    chunk = next(c for c in (1024, 512, 256, 128, 64, 32, 16, 8)
                 if N % c == 0)
    nb = N // chunk
    grid = (1, nb)
    nbj = grid[1]

    sub2d = jnp.broadcast_to(sub_idx.reshape(1, cutoff), (8, cutoff))
    y2d = y_labels.reshape(N, 1)

    grads, logits = pl.pallas_call(
        functools.partial(_fused_kernel, feat_dim=D, chunk=chunk),
        out_shape=(jax.ShapeDtypeStruct((N, cutoff), jnp.float32),
                   jax.ShapeDtypeStruct((N, C), jnp.float32)),
        grid_spec=pltpu.PrefetchScalarGridSpec(
            num_scalar_prefetch=0,
            grid=grid,
            in_specs=[
                pl.BlockSpec(memory_space=pl.ANY),
                pl.BlockSpec((C, D), lambda i, j: (0, 0)),
                pl.BlockSpec((chunk, 1), lambda i, j: (i * nbj + j, 0)),
                pl.BlockSpec((8, cutoff), lambda i, j: (0, 0)),
            ],
            out_specs=[
                pl.BlockSpec((chunk, cutoff), lambda i, j: (i * nbj + j, 0)),
                pl.BlockSpec((chunk, C), lambda i, j: (i * nbj + j, 0)),
            ],
            scratch_shapes=[pltpu.VMEM((nb, chunk, D), jnp.float32),
                            pltpu.VMEM((D, cutoff), jnp.bfloat16),
                            pltpu.VMEM((C, cutoff), jnp.bfloat16),
                            pltpu.SemaphoreType.DMA((nb,))]),
        compiler_params=pltpu.CompilerParams(
            dimension_semantics=("parallel", "arbitrary"),
            vmem_limit_bytes=_VMEM_LIMIT),
    )(x_flat, w, y2d, sub2d)
    return grads, logits
